# Initial kernel scaffold; baseline (speedup 1.0000x reference)
import jax
import jax.numpy as jnp
from jax.experimental import pallas as pl


def kernel(feat, demo, a_p, Wgcn1, bgcn1, Wgcn12, bgcn12, theta1, b1, theta2, b2, ln_w, ln_b):
    raise NotImplementedError("write your pallas kernel here")


# speedup vs baseline: 4.0797x; 4.0797x over previous
import functools

import jax
import jax.numpy as jnp
from jax import lax
from jax.experimental import pallas as pl

K = 10
BM = 256


def _stats_kernel(dd, x_ref, m_ref, v_ref, ss_ref):
    x = x_ref[...]
    s = jnp.sum(x, axis=1)
    ss = jnp.sum(x * x, axis=1)
    m = s / dd
    m_ref[0, :] = m
    v_ref[0, :] = ss - dd * m * m
    ss_ref[0, :] = ss


def _corrdist_kernel(dd, x_i_ref, x_j_ref, m_i_ref, v_i_ref, ss_i_ref,
                     m_j_ref, v_j_ref, ss_j_ref, comb_ref,
                     corr_ref, dist_ref, dsum_ref):
    j = pl.program_id(1)
    gm = lax.dot_general(x_i_ref[...], x_j_ref[...],
                         (((1,), (1,)), ((), ())),
                         preferred_element_type=jnp.float32)
    mi = m_i_ref[0, :][:, None]
    vi = v_i_ref[0, :][:, None]
    ssi = ss_i_ref[0, :][:, None]
    corr = (gm - dd * (mi * m_j_ref[...])) / jnp.sqrt(vi * v_j_ref[...])
    corr = jnp.clip(corr, -1.0, 1.0)
    corr = jnp.maximum(corr, 0.0) * comb_ref[...]
    corr_ref[...] = corr
    d2 = ssi + ss_j_ref[...] - 2.0 * gm
    dist_ref[...] = jnp.sqrt(jnp.maximum(d2, 0.0))

    @pl.when(j == 0)
    def _():
        dsum_ref[...] = jnp.zeros_like(dsum_ref)

    dsum_ref[0, :] += jnp.sum(corr, axis=1)


def _topk_kernel(n, x_ref, idx_ref):
    d = x_ref[...]
    bm = d.shape[0]
    iota = lax.broadcasted_iota(jnp.int32, (bm, n), 1)
    for t in range(K):
        m = jnp.min(d, axis=1, keepdims=True)
        amin = jnp.min(jnp.where(d == m, iota, n), axis=1)
        idx_ref[t, :] = amin
        d = jnp.where(iota == amin[:, None], jnp.float32(jnp.inf), d)


def _hist_kernel(idx_ref, dv_ref):
    jb = pl.program_id(0)
    bm = dv_ref.shape[1]
    jg = jb * bm + lax.broadcasted_iota(jnp.int32, (bm, 1), 0)
    acc = jnp.zeros((bm,), jnp.float32)
    for t in range(K):
        row = idx_ref[t, :][None, :]
        acc += jnp.sum((row == jg).astype(jnp.float32), axis=1)
    dv_ref[0, :] = acc


def _hbuild_kernel(idx_ref, ht_ref):
    jb = pl.program_id(1)
    bm = ht_ref.shape[0]
    jg = jb * bm + lax.broadcasted_iota(jnp.int32, (bm, bm), 1)
    acc = jnp.zeros((bm, bm), jnp.bool_)
    for t in range(K):
        acc = acc | (idx_ref[t, :][:, None] == jg)
    ht_ref[...] = acc.astype(jnp.float32)


def _mm_kernel(act, has_bias, *refs):
    if has_bias:
        x_ref, w_ref, b_ref, o_ref = refs
    else:
        x_ref, w_ref, o_ref = refs
        b_ref = None
    o = lax.dot_general(x_ref[...], w_ref[...], (((1,), (0,)), ((), ())),
                        preferred_element_type=jnp.float32)
    if b_ref is not None:
        o = o + b_ref[...]
    if act:
        o = jnp.maximum(o, 0.0)
    o_ref[...] = o


def _adj_kernel(act, corr_ref, dsum_i_ref, dsumT_ref, v_ref, b_ref, o_ref):
    vs = lax.rsqrt(dsumT_ref[...]) * v_ref[...]
    acc = lax.dot_general(corr_ref[...], vs, (((1,), (0,)), ((), ())),
                          preferred_element_type=jnp.float32)
    o = lax.rsqrt(dsum_i_ref[0, :])[:, None] * acc + b_ref[...]
    if act:
        o = jnp.maximum(o, 0.0)
    o_ref[...] = o


def _hta_kernel(ht_ref, dvT_ref, x_ref, o_ref):
    zs = lax.rsqrt(dvT_ref[...]) * x_ref[...]
    o_ref[...] = lax.dot_general(ht_ref[...], zs, (((1,), (0,)), ((), ())),
                                 preferred_element_type=jnp.float32)


def _htta_kernel(ht_ref, bmat_ref, dv_i_ref, o_ref):
    c = lax.dot_general(ht_ref[...], bmat_ref[...], (((0,), (0,)), ((), ())),
                        preferred_element_type=jnp.float32)
    x = lax.rsqrt(dv_i_ref[...]) * c * (1.0 / K)
    o_ref[...] = jnp.maximum(x, 0.0)


def _final_kernel(ht_ref, bmat_ref, dv_i_ref, pair_ref, lnw_ref, lnb_ref,
                  o_ref):
    c = lax.dot_general(ht_ref[...], bmat_ref[...], (((0,), (0,)), ((), ())),
                        preferred_element_type=jnp.float32)
    x2 = lax.rsqrt(dv_i_ref[...]) * c * (1.0 / K)
    fph = x2 + pair_ref[...]
    mu = jnp.mean(fph, axis=1, keepdims=True)
    var = jnp.mean((fph - mu) ** 2, axis=1, keepdims=True)
    o_ref[...] = (fph - mu) / jnp.sqrt(var + 1e-6) * lnw_ref[...] + lnb_ref[...]


def kernel(feat, demo, a_p, Wgcn1, bgcn1, Wgcn12, bgcn12, theta1, b1,
           theta2, b2, ln_w, ln_b):
    if feat.ndim == 3:
        feat = feat[0]
    if demo.ndim == 3:
        demo = demo[0]
    n, d = feat.shape
    h = theta1.shape[1]
    dd = float(d)
    bm = min(BM, n)
    g = n // bm
    f32 = jnp.float32

    comb = ((demo[:, 0] == demo[:, 0]) & (demo[:, 1] == demo[:, 1])
            & (demo[:, 2] == demo[:, 2])
            & (jnp.abs(demo[:, 3] - demo[:, 3]) < 5)
            & (jnp.abs(demo[:, 4] - demo[:, 4]) < 5))
    comb = comb.astype(f32)[None, :]

    row_1n = lambda: pl.BlockSpec((1, bm), lambda i, j: (0, j))
    full = lambda a: pl.BlockSpec(a.shape, lambda *_: (0,) * a.ndim)

    m, v, ss = pl.pallas_call(
        functools.partial(_stats_kernel, dd),
        grid=(g,),
        in_specs=[pl.BlockSpec((bm, d), lambda i: (i, 0))],
        out_specs=[pl.BlockSpec((1, bm), lambda i: (0, i))] * 3,
        out_shape=[jax.ShapeDtypeStruct((1, n), f32)] * 3,
    )(feat)

    corr, dist, dsum = pl.pallas_call(
        functools.partial(_corrdist_kernel, dd),
        grid=(g, g),
        in_specs=[
            pl.BlockSpec((bm, d), lambda i, j: (i, 0)),
            pl.BlockSpec((bm, d), lambda i, j: (j, 0)),
            pl.BlockSpec((1, bm), lambda i, j: (0, i)),
            pl.BlockSpec((1, bm), lambda i, j: (0, i)),
            pl.BlockSpec((1, bm), lambda i, j: (0, i)),
            row_1n(), row_1n(), row_1n(), row_1n(),
        ],
        out_specs=[
            pl.BlockSpec((bm, bm), lambda i, j: (i, j)),
            pl.BlockSpec((bm, bm), lambda i, j: (i, j)),
            pl.BlockSpec((1, bm), lambda i, j: (0, i)),
        ],
        out_shape=[
            jax.ShapeDtypeStruct((n, n), f32),
            jax.ShapeDtypeStruct((n, n), f32),
            jax.ShapeDtypeStruct((1, n), f32),
        ],
    )(feat, feat, m, v, ss, m, v, ss, comb)

    idxT = pl.pallas_call(
        functools.partial(_topk_kernel, n),
        grid=(g,),
        in_specs=[pl.BlockSpec((bm, n), lambda i: (i, 0))],
        out_specs=pl.BlockSpec((K, bm), lambda i: (0, i)),
        out_shape=jax.ShapeDtypeStruct((K, n), jnp.int32),
    )(dist)

    dv = pl.pallas_call(
        _hist_kernel,
        grid=(g,),
        in_specs=[pl.BlockSpec((K, n), lambda j: (0, 0))],
        out_specs=pl.BlockSpec((1, bm), lambda j: (0, j)),
        out_shape=jax.ShapeDtypeStruct((1, n), f32),
    )(idxT)

    ht = pl.pallas_call(
        _hbuild_kernel,
        grid=(g, g),
        in_specs=[pl.BlockSpec((K, bm), lambda i, j: (0, i))],
        out_specs=pl.BlockSpec((bm, bm), lambda i, j: (i, j)),
        out_shape=jax.ShapeDtypeStruct((n, n), f32),
    )(idxT)

    def mm(x, w, b=None, act=False):
        nr = x.shape[0]
        gg = nr // bm
        if b is None:
            return pl.pallas_call(
                functools.partial(_mm_kernel, act, False),
                grid=(gg,),
                in_specs=[pl.BlockSpec((bm, x.shape[1]), lambda i: (i, 0)),
                          full(w)],
                out_specs=pl.BlockSpec((bm, w.shape[1]), lambda i: (i, 0)),
                out_shape=jax.ShapeDtypeStruct((nr, w.shape[1]), f32),
            )(x, w)
        b2d = b.reshape(1, -1)
        return pl.pallas_call(
            functools.partial(_mm_kernel, act, True),
            grid=(gg,),
            in_specs=[pl.BlockSpec((bm, x.shape[1]), lambda i: (i, 0)),
                      full(w), full(b2d)],
            out_specs=pl.BlockSpec((bm, w.shape[1]), lambda i: (i, 0)),
            out_shape=jax.ShapeDtypeStruct((nr, w.shape[1]), f32),
        )(x, w, b2d)

    dsumT = dsum.reshape(n, 1)

    def adj_mm(vmat, b, act):
        b2d = b.reshape(1, -1)
        return pl.pallas_call(
            functools.partial(_adj_kernel, act),
            grid=(g,),
            in_specs=[
                pl.BlockSpec((bm, n), lambda i: (i, 0)),
                pl.BlockSpec((1, bm), lambda i: (0, i)),
                full(dsumT), full(vmat), full(b2d),
            ],
            out_specs=pl.BlockSpec((bm, h), lambda i: (i, 0)),
            out_shape=jax.ShapeDtypeStruct((n, h), f32),
        )(corr, dsum, dsumT, vmat, b2d)

    support = mm(feat, Wgcn1)
    out1 = adj_mm(support, bgcn1, act=True)
    pair = adj_mm(mm(out1, Wgcn12), bgcn12, act=False)

    dvT = dv.reshape(n, 1)

    def hta(x):
        return pl.pallas_call(
            _hta_kernel,
            grid=(g,),
            in_specs=[pl.BlockSpec((bm, n), lambda i: (i, 0)),
                      full(dvT), full(x)],
            out_specs=pl.BlockSpec((bm, h), lambda i: (i, 0)),
            out_shape=jax.ShapeDtypeStruct((n, h), f32),
        )(ht, dvT, x)

    x1 = mm(feat, theta1, b1)
    bmat1 = hta(x1)
    xl1 = pl.pallas_call(
        _htta_kernel,
        grid=(g,),
        in_specs=[pl.BlockSpec((n, bm), lambda jb: (0, jb)),
                  full(bmat1),
                  pl.BlockSpec((bm, 1), lambda jb: (jb, 0))],
        out_specs=pl.BlockSpec((bm, h), lambda jb: (jb, 0)),
        out_shape=jax.ShapeDtypeStruct((n, h), f32),
    )(ht, bmat1, dvT)

    x2 = mm(xl1, theta2, b2)
    bmat2 = hta(x2)

    lnw2d = ln_w.reshape(1, -1)
    lnb2d = ln_b.reshape(1, -1)
    outn = pl.pallas_call(
        _final_kernel,
        grid=(g,),
        in_specs=[pl.BlockSpec((n, bm), lambda jb: (0, jb)),
                  full(bmat2),
                  pl.BlockSpec((bm, 1), lambda jb: (jb, 0)),
                  pl.BlockSpec((bm, h), lambda jb: (jb, 0)),
                  full(lnw2d), full(lnb2d)],
        out_specs=pl.BlockSpec((bm, h), lambda jb: (jb, 0)),
        out_shape=jax.ShapeDtypeStruct((n, h), f32),
    )(ht, bmat2, dvT, pair, lnw2d, lnb2d)

    return outn[None]
